# baseline (speedup 1.0000x reference)
import functools

import jax
import jax.numpy as jnp
from jax import lax
from jax.experimental import pallas as pl
from jax.experimental.pallas import tpu as pltpu
from jax.experimental.pallas import tpu_sc as plsc

N = 10000
E = 160000
K = 4
DIM = 4
C = 256
H = 128
NC = 2
NS = 16
NW = NC * NS
B = 80
EPT = E // NS
NCHUNK = EPT // B
NP = 10240
HR = NP // H
STRIPE = NP // NS
ZR = 128
EPW = E // NW

_EPS = 1e-15
_ROW_BLK = 400


def _dense_body(x_ref, gcat_ref, root_ref, lin_ref, bias_ref, xg_ref, den_ref):
    x = x_ref[...]
    xg_ref[...] = lax.dot_general(
        x, gcat_ref[...], (((1,), (1,)), ((), ())),
        preferred_element_type=jnp.float32).astype(jnp.bfloat16)
    den = lax.dot_general(x, root_ref[...], (((1,), (1,)), ((), ())),
                          preferred_element_type=jnp.float32)
    den += lax.dot_general(x, lin_ref[...], (((1,), (1,)), ((), ())),
                           preferred_element_type=jnp.float32)
    den_ref[...] = den + bias_ref[...]


def _dense_stage(x, g, root, lin, bias):
    gcat = g.reshape(C, 2, 2, NC, H).transpose(3, 0, 1, 2, 4)
    gcat = (gcat.reshape(NC, C, 2, 2, 4, 2, 16).swapaxes(5, 6)
            .reshape(NC, C, 2, 2 * H))
    xg, den = pl.pallas_call(
        _dense_body,
        grid=(N // _ROW_BLK,),
        in_specs=[
            pl.BlockSpec((_ROW_BLK, C), lambda i: (i, 0)),
            pl.BlockSpec((NC, C, 2, 2 * H), lambda i: (0, 0, 0, 0)),
            pl.BlockSpec((C, C), lambda i: (0, 0)),
            pl.BlockSpec((C, C), lambda i: (0, 0)),
            pl.BlockSpec((1, C), lambda i: (0, 0)),
        ],
        out_specs=[
            pl.BlockSpec((_ROW_BLK, NC, 2, 2 * H), lambda i: (i, 0, 0, 0)),
            pl.BlockSpec((_ROW_BLK, C), lambda i: (i, 0)),
        ],
        out_shape=[
            jax.ShapeDtypeStruct((N, NC, 2, 2 * H), jnp.bfloat16),
            jax.ShapeDtypeStruct((N, C), jnp.float32),
        ],
    )(x, gcat, root, lin, bias.reshape(1, C))
    xgi = lax.bitcast_convert_type(
        xg.reshape(N * NC * 2, H, 2), jnp.int32)
    return xgi, den


REC = 6 * B


def _sc_body(xg_hbm, rec_hbm, mu_hbm, sg_hbm, out_hbm,
             mu_v, w_v, idx_v, dst_v, gau_v, rec_v, r0_v, r1_v, msg_v,
             shared, semr, sems):
    cid = lax.axis_index("c")
    sid = lax.axis_index("s")

    pltpu.sync_copy(mu_hbm, mu_v)
    pltpu.sync_copy(sg_hbm, w_v)
    s = w_v[...]
    muv = mu_v[...]
    wv = -0.5 / (_EPS + s * s)

    def zrow(r, _):
        for i in range(H // 16):
            msg_v[r, pl.ds(i * 16, 16)] = jnp.zeros((16,), jnp.float32)
        return _
    lax.fori_loop(0, B, zrow, 0)
    for i in range(STRIPE // B):
        pltpu.sync_copy(msg_v, shared.at[pl.ds(sid * STRIPE + i * B, B)])

    plsc.subcore_barrier()

    pltpu.async_copy(rec_hbm.at[pl.ds(sid * NCHUNK * REC, REC)],
                     rec_v.at[pl.ds(0, REC)], semr)
    rbufs = (r0_v, r1_v)

    def chunk(c, _):
        p = lax.rem(c, 2)
        rbase = p * REC
        pltpu.make_async_copy(rec_hbm.at[pl.ds(0, REC)],
                              rec_v.at[pl.ds(rbase, REC)], semr).wait()

        for i in range(B // 16):
            sl = pl.ds(i * 16, 16)
            i0 = rec_v[pl.ds(rbase + i * 16, 16)] * 4 + cid * 2
            idx_v[pl.ds(i * 16, 16)] = i0
            idx_v[pl.ds(B + i * 16, 16)] = i0 + 1
            dst_v[sl] = rec_v[pl.ds(rbase + B + i * 16, 16)]

        for h in range(2):
            pltpu.async_copy(xg_hbm.at[idx_v.at[pl.ds(h * B, B)]],
                             rbufs[h], sems.at[h])

        @pl.when(c < NCHUNK - 1)
        def _prefetch():
            nxt = sid * NCHUNK + c + 1
            pltpu.async_copy(rec_hbm.at[pl.ds(nxt * REC, REC)],
                             rec_v.at[pl.ds((1 - p) * REC, REC)], semr)

        for k in range(K):
            for i in range(B // 16):
                sl = pl.ds(i * 16, 16)
                acc = jnp.zeros((16,), jnp.float32)
                for d in range(DIM):
                    t = plsc.bitcast(
                        rec_v[pl.ds(rbase + (2 + d) * B + i * 16, 16)],
                        jnp.float32) - muv[k * DIM + d]
                    acc += t * t * wv[k * DIM + d]
                gau_v[pl.ds(k * B + i * 16, 16)] = jnp.exp(acc)

        for h in range(2):
            pltpu.make_async_copy(xg_hbm.at[idx_v.at[pl.ds(h * B, B)]],
                                  rbufs[h], sems.at[h]).wait()

        ifmt = plsc.PackFormat.INTERLEAVED

        def edge(b, _):
            bs = jnp.full((16,), b, jnp.int32)
            gp = []
            for k in range(K):
                gk = plsc.load_gather(gau_v, [bs + k * B])
                gp.append(plsc.pack(gk, gk, format=ifmt))
            for q in range(H // 32):
                v = (gp[0] * plsc.bitcast(r0_v[b, pl.ds(q * 16, 16)],
                                          jnp.bfloat16)
                     + gp[1] * plsc.bitcast(r0_v[b, pl.ds(64 + q * 16, 16)],
                                            jnp.bfloat16)
                     + gp[2] * plsc.bitcast(r1_v[b, pl.ds(q * 16, 16)],
                                            jnp.bfloat16)
                     + gp[3] * plsc.bitcast(r1_v[b, pl.ds(64 + q * 16, 16)],
                                            jnp.bfloat16))
                va, vb = plsc.unpack(v, format=ifmt)
                msg_v[b, pl.ds(q * 32, 16)] = va
                msg_v[b, pl.ds(q * 32 + 16, 16)] = vb
            return _
        lax.fori_loop(0, B, edge, 0)

        pltpu.sync_copy(msg_v, shared.at[dst_v], add=True)
        return _
    lax.fori_loop(0, NCHUNK, chunk, 0)

    plsc.subcore_barrier()
    pltpu.sync_copy(shared.at[pl.ds(sid * STRIPE, STRIPE)],
                    out_hbm.at[cid, pl.ds(sid * STRIPE, STRIPE)])


_sc_conv = functools.partial(
    pl.kernel,
    out_type=jax.ShapeDtypeStruct((NC, NP, H), jnp.float32),
    mesh=plsc.VectorSubcoreMesh(core_axis_name="c", subcore_axis_name="s", num_cores=NC, num_subcores=NS),
    compiler_params=pltpu.CompilerParams(needs_layout_passes=False),
    scratch_types=[
        pltpu.VMEM((16,), jnp.float32),
        pltpu.VMEM((16,), jnp.float32),
        pltpu.VMEM((2 * B,), jnp.int32),
        pltpu.VMEM((B,), jnp.int32),
        pltpu.VMEM((K * B,), jnp.float32),
        pltpu.VMEM((2 * REC,), jnp.int32),
        pltpu.VMEM((B, H), jnp.int32),
        pltpu.VMEM((B, H), jnp.int32),
        pltpu.VMEM((B, H), jnp.float32),
        pltpu.VMEM_SHARED((NP, H), jnp.float32),
        pltpu.SemaphoreType.DMA,
        pltpu.SemaphoreType.DMA((2,)),
    ],
)(_sc_body)


BC = 40
EPC = E // NC
EPTC = EPC // NS


def _cnt_body(dst_hbm, out_hbm, dst_v, one_v, shared):
    cid = lax.axis_index("c")
    sid = lax.axis_index("s")

    def zrow(r, _):
        for i in range(H // 16):
            one_v[r, pl.ds(i * 16, 16)] = jnp.zeros((16,), jnp.float32)
        return _
    lax.fori_loop(0, BC, zrow, 0)
    for i in range(STRIPE // BC):
        pltpu.sync_copy(one_v, shared.at[pl.ds(sid * STRIPE + i * BC, BC)])

    def orow(r, _):
        for i in range(H // 16):
            one_v[r, pl.ds(i * 16, 16)] = jnp.ones((16,), jnp.float32)
        return _
    lax.fori_loop(0, BC, orow, 0)

    plsc.subcore_barrier()

    def chunk(c, _):
        ebase = cid * EPC + sid * EPTC + c * BC
        pltpu.sync_copy(dst_hbm.at[pl.ds(ebase, BC)], dst_v)
        pltpu.sync_copy(one_v, shared.at[dst_v], add=True)
        return _
    lax.fori_loop(0, EPTC // BC, chunk, 0)

    plsc.subcore_barrier()
    pltpu.sync_copy(shared.at[pl.ds(sid * STRIPE, STRIPE)],
                    out_hbm.at[cid, pl.ds(sid * STRIPE, STRIPE)])


_sc_cnt = functools.partial(
    pl.kernel,
    out_type=jax.ShapeDtypeStruct((NC, NP, H), jnp.float32),
    mesh=plsc.VectorSubcoreMesh(core_axis_name="c", subcore_axis_name="s", num_cores=NC, num_subcores=NS),
    compiler_params=pltpu.CompilerParams(needs_layout_passes=False),
    scratch_types=[
        pltpu.VMEM((BC,), jnp.int32),
        pltpu.VMEM((BC, H), jnp.float32),
        pltpu.VMEM_SHARED((NP, H), jnp.float32),
    ],
)(_cnt_body)


def _bn_body(acc_ref, cnt_ref, den_ref, gamma_ref, beta_ref, out_ref, *, relu):
    summed = jnp.concatenate(
        [acc_ref[0, :N, :], acc_ref[1, :N, :]], axis=1)
    cnt = cnt_ref[0, :N, 0:1] + cnt_ref[1, :N, 0:1]
    t = summed / jnp.clip(cnt, 1.0, None) + den_ref[...]
    mean = jnp.mean(t, axis=0, keepdims=True)
    var = jnp.mean((t - mean) ** 2, axis=0, keepdims=True)
    h = (t - mean) * lax.rsqrt(var + 1e-5) * gamma_ref[...] + beta_ref[...]
    if relu:
        h = jnp.maximum(h, 0.0)
    out_ref[...] = h


def _bn_stage(acc, cnt, den, gamma, beta, relu):
    return pl.pallas_call(
        functools.partial(_bn_body, relu=relu),
        grid=(1,),
        in_specs=[
            pl.BlockSpec((NC, NP, H), lambda i: (0, 0, 0)),
            pl.BlockSpec((NC, NP, H), lambda i: (0, 0, 0)),
            pl.BlockSpec((N, C), lambda i: (0, 0)),
            pl.BlockSpec((1, C), lambda i: (0, 0)),
            pl.BlockSpec((1, C), lambda i: (0, 0)),
        ],
        out_specs=pl.BlockSpec((N, C), lambda i: (0, 0)),
        out_shape=jax.ShapeDtypeStruct((N, C), jnp.float32),
    )(acc, cnt, den, gamma.reshape(1, C), beta.reshape(1, C))


def _layer(x, rec, cnt, g, mu, sigma, root, bias, lin, gamma, beta, relu):
    xg, den = _dense_stage(x, g, root, lin, bias)
    acc = _sc_conv(xg, rec, mu.reshape(K * DIM), sigma.reshape(K * DIM))
    return _bn_stage(acc, cnt, den, gamma, beta, relu)


def kernel(vals, edges, pseudo, g0, mu0, sigma0, root0, bias0, lin0, gamma0,
           beta0, g1, mu1, sigma1, root1, bias1, lin1, gamma1, beta1):
    src = edges[0]
    dst = edges[1]
    pbits = lax.bitcast_convert_type(pseudo.T, jnp.int32)
    rec = jnp.concatenate([src[None], dst[None], pbits], axis=0)
    rec = rec.reshape(6, E // B, B).transpose(1, 0, 2).reshape(E // B * REC)
    cnt = _sc_cnt(dst)
    h = _layer(vals, rec, cnt, g0, mu0, sigma0, root0, bias0, lin0,
               gamma0, beta0, True)
    h = _layer(h, rec, cnt, g1, mu1, sigma1, root1, bias1, lin1,
               gamma1, beta1, False)
    return h


# speedup vs baseline: 5.2385x; 5.2385x over previous
import functools

import jax
import jax.numpy as jnp
from jax import lax
from jax.experimental import pallas as pl
from jax.experimental.pallas import tpu as pltpu
from jax.experimental.pallas import tpu_sc as plsc

N = 10000
E = 160000
K = 4
DIM = 4
C = 256
H = 128
NC = 2
NS = 16
NW = NC * NS
B = 80
EPT = E // NS
NCHUNK = EPT // B
NP = 10240
HR = NP // H
STRIPE = NP // NS
ZR = 128
EPW = E // NW

_EPS = 1e-15
_ROW_BLK = 400


def _dense_body(x_ref, gcat_ref, root_ref, lin_ref, bias_ref, xg_ref, den_ref):
    x = x_ref[...]
    y = lax.dot_general(x, gcat_ref[...], (((1,), (1,)), ((), ())),
                        preferred_element_type=jnp.float32)
    y = y.reshape(_ROW_BLK, NC, 2, 2, 2, 64)
    lo = lax.bitcast_convert_type(
        y[:, :, :, :, 0, :].astype(jnp.bfloat16), jnp.uint16)
    hi = lax.bitcast_convert_type(
        y[:, :, :, :, 1, :].astype(jnp.bfloat16), jnp.uint16)
    w = lo.astype(jnp.uint32) | (hi.astype(jnp.uint32) << 16)
    xg_ref[...] = lax.bitcast_convert_type(w, jnp.int32).reshape(
        _ROW_BLK, NC, 2, H)
    den = lax.dot_general(x, root_ref[...], (((1,), (1,)), ((), ())),
                          preferred_element_type=jnp.float32)
    den += lax.dot_general(x, lin_ref[...], (((1,), (1,)), ((), ())),
                           preferred_element_type=jnp.float32)
    den_ref[...] = den + bias_ref[...]


def _dense_stage(x, g, root, lin, bias):
    gcat = (g.reshape(C, 2, 2, NC, H).transpose(3, 0, 1, 2, 4)
            .reshape(NC, C, 2, 2 * H))
    xg, den = pl.pallas_call(
        _dense_body,
        grid=(N // _ROW_BLK,),
        in_specs=[
            pl.BlockSpec((_ROW_BLK, C), lambda i: (i, 0)),
            pl.BlockSpec((NC, C, 2, 2 * H), lambda i: (0, 0, 0, 0)),
            pl.BlockSpec((C, C), lambda i: (0, 0)),
            pl.BlockSpec((C, C), lambda i: (0, 0)),
            pl.BlockSpec((1, C), lambda i: (0, 0)),
        ],
        out_specs=[
            pl.BlockSpec((_ROW_BLK, NC, 2, H), lambda i: (i, 0, 0, 0)),
            pl.BlockSpec((_ROW_BLK, C), lambda i: (i, 0)),
        ],
        out_shape=[
            jax.ShapeDtypeStruct((N, NC, 2, H), jnp.int32),
            jax.ShapeDtypeStruct((N, C), jnp.float32),
        ],
    )(x, gcat, root, lin, bias.reshape(1, C))
    return xg.reshape(N * NC * 2, H), den


REC = 6 * B


def _sc_body(xg_hbm, rec_hbm, mu_hbm, sg_hbm, out_hbm,
             mu_v, w_v, idx_v, dst_v, gau_v, rec_v, r0_v, r1_v, msg_v,
             shared, semr, sems):
    cid = lax.axis_index("c")
    sid = lax.axis_index("s")

    pltpu.sync_copy(mu_hbm, mu_v)
    pltpu.sync_copy(sg_hbm, w_v)
    s = w_v[...]
    muv = mu_v[...]
    wv = -0.5 / (_EPS + s * s)

    def zrow(r, _):
        for i in range(H // 16):
            msg_v[r, pl.ds(i * 16, 16)] = jnp.zeros((16,), jnp.float32)
        return _
    lax.fori_loop(0, B, zrow, 0)
    for i in range(STRIPE // B):
        pltpu.sync_copy(msg_v, shared.at[pl.ds(sid * STRIPE + i * B, B)])

    plsc.subcore_barrier()

    pltpu.async_copy(rec_hbm.at[pl.ds(sid * NCHUNK * REC, REC)],
                     rec_v.at[pl.ds(0, REC)], semr)
    rbufs = (r0_v, r1_v)

    def chunk(c, _):
        p = lax.rem(c, 2)
        rbase = p * REC
        pltpu.make_async_copy(rec_hbm.at[pl.ds(0, REC)],
                              rec_v.at[pl.ds(rbase, REC)], semr).wait()

        for i in range(B // 16):
            sl = pl.ds(i * 16, 16)
            i0 = rec_v[pl.ds(rbase + i * 16, 16)] * 4 + cid * 2
            idx_v[pl.ds(i * 16, 16)] = i0
            idx_v[pl.ds(B + i * 16, 16)] = i0 + 1
            dst_v[sl] = rec_v[pl.ds(rbase + B + i * 16, 16)]

        for h in range(2):
            pltpu.async_copy(xg_hbm.at[idx_v.at[pl.ds(h * B, B)]],
                             rbufs[h], sems.at[h])

        @pl.when(c < NCHUNK - 1)
        def _prefetch():
            nxt = sid * NCHUNK + c + 1
            pltpu.async_copy(rec_hbm.at[pl.ds(nxt * REC, REC)],
                             rec_v.at[pl.ds((1 - p) * REC, REC)], semr)

        for k in range(K):
            for i in range(B // 16):
                sl = pl.ds(i * 16, 16)
                acc = jnp.zeros((16,), jnp.float32)
                for d in range(DIM):
                    t = plsc.bitcast(
                        rec_v[pl.ds(rbase + (2 + d) * B + i * 16, 16)],
                        jnp.float32) - muv[k * DIM + d]
                    acc += t * t * wv[k * DIM + d]
                gau_v[pl.ds(k * B + i * 16, 16)] = jnp.exp(acc)

        for h in range(2):
            pltpu.make_async_copy(xg_hbm.at[idx_v.at[pl.ds(h * B, B)]],
                                  rbufs[h], sems.at[h]).wait()

        ifmt = plsc.PackFormat.INTERLEAVED

        def edge(b, _):
            bs = jnp.full((16,), b, jnp.int32)
            gp = []
            for k in range(K):
                gk = plsc.load_gather(gau_v, [bs + k * B])
                gp.append(plsc.pack(gk, gk, format=ifmt))
            for q in range(H // 32):
                v = (gp[0] * plsc.bitcast(r0_v[b, pl.ds(q * 16, 16)],
                                          jnp.bfloat16)
                     + gp[1] * plsc.bitcast(r0_v[b, pl.ds(64 + q * 16, 16)],
                                            jnp.bfloat16)
                     + gp[2] * plsc.bitcast(r1_v[b, pl.ds(q * 16, 16)],
                                            jnp.bfloat16)
                     + gp[3] * plsc.bitcast(r1_v[b, pl.ds(64 + q * 16, 16)],
                                            jnp.bfloat16))
                va, vb = plsc.unpack(v, format=ifmt)
                msg_v[b, pl.ds(q * 16, 16)] = va
                msg_v[b, pl.ds(64 + q * 16, 16)] = vb
            return _
        lax.fori_loop(0, B, edge, 0)

        pltpu.sync_copy(msg_v, shared.at[dst_v], add=True)
        return _
    lax.fori_loop(0, NCHUNK, chunk, 0)

    plsc.subcore_barrier()
    pltpu.sync_copy(shared.at[pl.ds(sid * STRIPE, STRIPE)],
                    out_hbm.at[cid, pl.ds(sid * STRIPE, STRIPE)])


_sc_conv = functools.partial(
    pl.kernel,
    out_type=jax.ShapeDtypeStruct((NC, NP, H), jnp.float32),
    mesh=plsc.VectorSubcoreMesh(core_axis_name="c", subcore_axis_name="s", num_cores=NC, num_subcores=NS),
    compiler_params=pltpu.CompilerParams(needs_layout_passes=False),
    scratch_types=[
        pltpu.VMEM((16,), jnp.float32),
        pltpu.VMEM((16,), jnp.float32),
        pltpu.VMEM((2 * B,), jnp.int32),
        pltpu.VMEM((B,), jnp.int32),
        pltpu.VMEM((K * B,), jnp.float32),
        pltpu.VMEM((2 * REC,), jnp.int32),
        pltpu.VMEM((B, H), jnp.int32),
        pltpu.VMEM((B, H), jnp.int32),
        pltpu.VMEM((B, H), jnp.float32),
        pltpu.VMEM_SHARED((NP, H), jnp.float32),
        pltpu.SemaphoreType.DMA,
        pltpu.SemaphoreType.DMA((2,)),
    ],
)(_sc_body)


BC = 40
EPC = E // NC
EPTC = EPC // NS


def _cnt_body(dst_hbm, out_hbm, dst_v, one_v, shared):
    cid = lax.axis_index("c")
    sid = lax.axis_index("s")

    def zrow(r, _):
        for i in range(H // 16):
            one_v[r, pl.ds(i * 16, 16)] = jnp.zeros((16,), jnp.float32)
        return _
    lax.fori_loop(0, BC, zrow, 0)
    for i in range(STRIPE // BC):
        pltpu.sync_copy(one_v, shared.at[pl.ds(sid * STRIPE + i * BC, BC)])

    def orow(r, _):
        for i in range(H // 16):
            one_v[r, pl.ds(i * 16, 16)] = jnp.ones((16,), jnp.float32)
        return _
    lax.fori_loop(0, BC, orow, 0)

    plsc.subcore_barrier()

    def chunk(c, _):
        ebase = cid * EPC + sid * EPTC + c * BC
        pltpu.sync_copy(dst_hbm.at[pl.ds(ebase, BC)], dst_v)
        pltpu.sync_copy(one_v, shared.at[dst_v], add=True)
        return _
    lax.fori_loop(0, EPTC // BC, chunk, 0)

    plsc.subcore_barrier()
    pltpu.sync_copy(shared.at[pl.ds(sid * STRIPE, STRIPE)],
                    out_hbm.at[cid, pl.ds(sid * STRIPE, STRIPE)])


_sc_cnt = functools.partial(
    pl.kernel,
    out_type=jax.ShapeDtypeStruct((NC, NP, H), jnp.float32),
    mesh=plsc.VectorSubcoreMesh(core_axis_name="c", subcore_axis_name="s", num_cores=NC, num_subcores=NS),
    compiler_params=pltpu.CompilerParams(needs_layout_passes=False),
    scratch_types=[
        pltpu.VMEM((BC,), jnp.int32),
        pltpu.VMEM((BC, H), jnp.float32),
        pltpu.VMEM_SHARED((NP, H), jnp.float32),
    ],
)(_cnt_body)


def _bn_body(acc_ref, cnt_ref, den_ref, gamma_ref, beta_ref, out_ref, *, relu):
    summed = jnp.concatenate(
        [acc_ref[0, :N, :], acc_ref[1, :N, :]], axis=1)
    cnt = cnt_ref[0, :N, 0:1] + cnt_ref[1, :N, 0:1]
    t = summed / jnp.clip(cnt, 1.0, None) + den_ref[...]
    mean = jnp.mean(t, axis=0, keepdims=True)
    var = jnp.mean((t - mean) ** 2, axis=0, keepdims=True)
    h = (t - mean) * lax.rsqrt(var + 1e-5) * gamma_ref[...] + beta_ref[...]
    if relu:
        h = jnp.maximum(h, 0.0)
    out_ref[...] = h


def _bn_stage(acc, cnt, den, gamma, beta, relu):
    return pl.pallas_call(
        functools.partial(_bn_body, relu=relu),
        grid=(1,),
        in_specs=[
            pl.BlockSpec((NC, NP, H), lambda i: (0, 0, 0)),
            pl.BlockSpec((NC, NP, H), lambda i: (0, 0, 0)),
            pl.BlockSpec((N, C), lambda i: (0, 0)),
            pl.BlockSpec((1, C), lambda i: (0, 0)),
            pl.BlockSpec((1, C), lambda i: (0, 0)),
        ],
        out_specs=pl.BlockSpec((N, C), lambda i: (0, 0)),
        out_shape=jax.ShapeDtypeStruct((N, C), jnp.float32),
    )(acc, cnt, den, gamma.reshape(1, C), beta.reshape(1, C))


def _layer(x, rec, cnt, g, mu, sigma, root, bias, lin, gamma, beta, relu):
    xg, den = _dense_stage(x, g, root, lin, bias)
    acc = _sc_conv(xg, rec, mu.reshape(K * DIM), sigma.reshape(K * DIM))
    return _bn_stage(acc, cnt, den, gamma, beta, relu)


def kernel(vals, edges, pseudo, g0, mu0, sigma0, root0, bias0, lin0, gamma0,
           beta0, g1, mu1, sigma1, root1, bias1, lin1, gamma1, beta1):
    src = edges[0]
    dst = edges[1]
    pbits = lax.bitcast_convert_type(pseudo.T, jnp.int32)
    rec = jnp.concatenate([src[None], dst[None], pbits], axis=0)
    rec = rec.reshape(6, E // B, B).transpose(1, 0, 2).reshape(E // B * REC)
    cnt = _sc_cnt(dst)
    h = _layer(vals, rec, cnt, g0, mu0, sigma0, root0, bias0, lin0,
               gamma0, beta0, True)
    h = _layer(h, rec, cnt, g1, mu1, sigma1, root1, bias1, lin1,
               gamma1, beta1, False)
    return h


# speedup vs baseline: 6.9241x; 1.3218x over previous
import functools

import jax
import jax.numpy as jnp
from jax import lax
from jax.experimental import pallas as pl
from jax.experimental.pallas import tpu as pltpu
from jax.experimental.pallas import tpu_sc as plsc

N = 10000
E = 160000
K = 4
DIM = 4
C = 256
H = 128
NC = 2
NS = 16
NW = NC * NS
B = 80
EPT = E // NS
NCHUNK = EPT // B
NP = 10240
HR = NP // H
STRIPE = NP // NS
ZR = 128
EPW = E // NW

_EPS = 1e-15
_ROW_BLK = 400


def _dense_body(x_ref, gcat_ref, root_ref, lin_ref, bias_ref, xg_ref, den_ref):
    x = x_ref[...]
    y = lax.dot_general(x, gcat_ref[...], (((1,), (1,)), ((), ())),
                        preferred_element_type=jnp.float32)
    y = y.reshape(_ROW_BLK, NC, 2, 2, 2, 64)
    lo = lax.bitcast_convert_type(
        y[:, :, :, :, 0, :].astype(jnp.bfloat16), jnp.uint16)
    hi = lax.bitcast_convert_type(
        y[:, :, :, :, 1, :].astype(jnp.bfloat16), jnp.uint16)
    w = lo.astype(jnp.uint32) | (hi.astype(jnp.uint32) << 16)
    xg_ref[...] = lax.bitcast_convert_type(w, jnp.float32).reshape(
        _ROW_BLK, NC, 2, H)
    den = lax.dot_general(x, root_ref[...], (((1,), (1,)), ((), ())),
                          preferred_element_type=jnp.float32)
    den += lax.dot_general(x, lin_ref[...], (((1,), (1,)), ((), ())),
                           preferred_element_type=jnp.float32)
    den_ref[...] = den + bias_ref[...]


def _dense_stage(x, g, root, lin, bias):
    gcat = (g.reshape(C, 2, 2, NC, H).transpose(3, 0, 1, 2, 4)
            .reshape(NC, C, 2, 2 * H))
    xg, den = pl.pallas_call(
        _dense_body,
        grid=(N // _ROW_BLK,),
        in_specs=[
            pl.BlockSpec((_ROW_BLK, C), lambda i: (i, 0)),
            pl.BlockSpec((NC, C, 2, 2 * H), lambda i: (0, 0, 0, 0)),
            pl.BlockSpec((C, C), lambda i: (0, 0)),
            pl.BlockSpec((C, C), lambda i: (0, 0)),
            pl.BlockSpec((1, C), lambda i: (0, 0)),
        ],
        out_specs=[
            pl.BlockSpec((_ROW_BLK, NC, 2, H), lambda i: (i, 0, 0, 0)),
            pl.BlockSpec((_ROW_BLK, C), lambda i: (i, 0)),
        ],
        out_shape=[
            jax.ShapeDtypeStruct((N, NC, 2, H), jnp.float32),
            jax.ShapeDtypeStruct((N, C), jnp.float32),
        ],
    )(x, gcat, root, lin, bias.reshape(1, C))
    return xg.reshape(N * NC * 2, H), den


REC = 6 * B


def _sc_body(xg_hbm, rec_hbm, mu_hbm, sg_hbm, out_hbm,
             mu_v, w_v, idxa_v, idxb_v, dsta_v, dstb_v, gau_v, rec_v,
             r0a_v, r1a_v, r0b_v, r1b_v, shared, semr, sems):
    cid = lax.axis_index("c")
    sid = lax.axis_index("s")

    pltpu.sync_copy(mu_hbm, mu_v)
    pltpu.sync_copy(sg_hbm, w_v)
    s = w_v[...]
    muv = mu_v[...]
    wv = -0.5 / (_EPS + s * s)

    def zrow(r, _):
        for i in range(H // 16):
            r0a_v[r, pl.ds(i * 16, 16)] = jnp.zeros((16,), jnp.float32)
        return _
    lax.fori_loop(0, B, zrow, 0)
    for i in range(STRIPE // B):
        pltpu.sync_copy(r0a_v, shared.at[pl.ds(sid * STRIPE + i * B, B)])

    plsc.subcore_barrier()

    bufs = ((r0a_v, r1a_v), (r0b_v, r1b_v))
    idxs = (idxa_v, idxb_v)
    dsts = (dsta_v, dstb_v)
    ifmt = plsc.PackFormat.INTERLEAVED

    def decode_fire(cc, pb):
        rb = pb * REC
        for i in range(B // 16):
            sl = pl.ds(i * 16, 16)
            i0 = rec_v[pl.ds(rb + i * 16, 16)] * 4 + cid * 2
            idxs[pb][pl.ds(i * 16, 16)] = i0
            idxs[pb][pl.ds(B + i * 16, 16)] = i0 + 1
            dsts[pb][sl] = rec_v[pl.ds(rb + B + i * 16, 16)]
        for h in range(2):
            pltpu.async_copy(xg_hbm.at[idxs[pb].at[pl.ds(h * B, B)]],
                             bufs[pb][h], sems.at[2 * pb + h])

    pltpu.sync_copy(rec_hbm.at[pl.ds(sid * NCHUNK * REC, REC)],
                    rec_v.at[pl.ds(0, REC)])
    decode_fire(0, 0)
    pltpu.async_copy(rec_hbm.at[pl.ds((sid * NCHUNK + 1) * REC, REC)],
                     rec_v.at[pl.ds(REC, REC)], semr)

    def stage(c, pb):
        qb = 1 - pb
        cur0, cur1 = bufs[pb]

        @pl.when(c < NCHUNK - 1)
        def _df():
            pltpu.make_async_copy(rec_hbm.at[pl.ds(0, REC)],
                                  rec_v.at[pl.ds(qb * REC, REC)], semr).wait()
            decode_fire(c + 1, qb)

        rb = pb * REC
        for k in range(K):
            for i in range(B // 16):
                sl = pl.ds(i * 16, 16)
                acc = jnp.zeros((16,), jnp.float32)
                for d in range(DIM):
                    t = plsc.bitcast(
                        rec_v[pl.ds(rb + (2 + d) * B + i * 16, 16)],
                        jnp.float32) - muv[k * DIM + d]
                    acc += t * t * wv[k * DIM + d]
                gau_v[pl.ds(k * B + i * 16, 16)] = jnp.exp(acc)

        @pl.when(c < NCHUNK - 2)
        def _pf():
            nxt = sid * NCHUNK + c + 2
            pltpu.async_copy(rec_hbm.at[pl.ds(nxt * REC, REC)],
                             rec_v.at[pl.ds(pb * REC, REC)], semr)

        for h in range(2):
            pltpu.make_async_copy(xg_hbm.at[idxs[pb].at[pl.ds(h * B, B)]],
                                  bufs[pb][h], sems.at[2 * pb + h]).wait()

        def edge(b, _):
            bs = jnp.full((16,), b, jnp.int32)
            gp = []
            for k in range(K):
                gk = plsc.load_gather(gau_v, [bs + k * B])
                gp.append(plsc.pack(gk, gk, format=ifmt))
            for q in range(H // 32):
                v = (gp[0] * plsc.bitcast(cur0[b, pl.ds(q * 16, 16)],
                                          jnp.bfloat16)
                     + gp[1] * plsc.bitcast(cur0[b, pl.ds(64 + q * 16, 16)],
                                            jnp.bfloat16)
                     + gp[2] * plsc.bitcast(cur1[b, pl.ds(q * 16, 16)],
                                            jnp.bfloat16)
                     + gp[3] * plsc.bitcast(cur1[b, pl.ds(64 + q * 16, 16)],
                                            jnp.bfloat16))
                va, vb = plsc.unpack(v, format=ifmt)
                cur0[b, pl.ds(q * 16, 16)] = va
                cur0[b, pl.ds(64 + q * 16, 16)] = vb
            return _
        lax.fori_loop(0, B, edge, 0)

        pltpu.sync_copy(cur0, shared.at[dsts[pb]], add=True)

    def chunk(c, _):
        p = lax.rem(c, 2)

        @pl.when(p == 0)
        def _even():
            stage(c, 0)

        @pl.when(p == 1)
        def _odd():
            stage(c, 1)
        return _
    lax.fori_loop(0, NCHUNK, chunk, 0)

    plsc.subcore_barrier()
    pltpu.sync_copy(shared.at[pl.ds(sid * STRIPE, STRIPE)],
                    out_hbm.at[cid, pl.ds(sid * STRIPE, STRIPE)])


_sc_conv = functools.partial(
    pl.kernel,
    out_type=jax.ShapeDtypeStruct((NC, NP, H), jnp.float32),
    mesh=plsc.VectorSubcoreMesh(core_axis_name="c", subcore_axis_name="s", num_cores=NC, num_subcores=NS),
    compiler_params=pltpu.CompilerParams(needs_layout_passes=False),
    scratch_types=[
        pltpu.VMEM((16,), jnp.float32),
        pltpu.VMEM((16,), jnp.float32),
        pltpu.VMEM((2 * B,), jnp.int32),
        pltpu.VMEM((2 * B,), jnp.int32),
        pltpu.VMEM((B,), jnp.int32),
        pltpu.VMEM((B,), jnp.int32),
        pltpu.VMEM((K * B,), jnp.float32),
        pltpu.VMEM((2 * REC,), jnp.int32),
        pltpu.VMEM((B, H), jnp.float32),
        pltpu.VMEM((B, H), jnp.float32),
        pltpu.VMEM((B, H), jnp.float32),
        pltpu.VMEM((B, H), jnp.float32),
        pltpu.VMEM_SHARED((NP, H), jnp.float32),
        pltpu.SemaphoreType.DMA,
        pltpu.SemaphoreType.DMA((4,)),
    ],
)(_sc_body)


BC = 40
EPC = E // NC
EPTC = EPC // NS


def _cnt_body(dst_hbm, out_hbm, dst_v, one_v, shared):
    cid = lax.axis_index("c")
    sid = lax.axis_index("s")

    def zrow(r, _):
        for i in range(H // 16):
            one_v[r, pl.ds(i * 16, 16)] = jnp.zeros((16,), jnp.float32)
        return _
    lax.fori_loop(0, BC, zrow, 0)
    for i in range(STRIPE // BC):
        pltpu.sync_copy(one_v, shared.at[pl.ds(sid * STRIPE + i * BC, BC)])

    def orow(r, _):
        for i in range(H // 16):
            one_v[r, pl.ds(i * 16, 16)] = jnp.ones((16,), jnp.float32)
        return _
    lax.fori_loop(0, BC, orow, 0)

    plsc.subcore_barrier()

    def chunk(c, _):
        ebase = cid * EPC + sid * EPTC + c * BC
        pltpu.sync_copy(dst_hbm.at[pl.ds(ebase, BC)], dst_v)
        pltpu.sync_copy(one_v, shared.at[dst_v], add=True)
        return _
    lax.fori_loop(0, EPTC // BC, chunk, 0)

    plsc.subcore_barrier()
    pltpu.sync_copy(shared.at[pl.ds(sid * STRIPE, STRIPE)],
                    out_hbm.at[cid, pl.ds(sid * STRIPE, STRIPE)])


_sc_cnt = functools.partial(
    pl.kernel,
    out_type=jax.ShapeDtypeStruct((NC, NP, H), jnp.float32),
    mesh=plsc.VectorSubcoreMesh(core_axis_name="c", subcore_axis_name="s", num_cores=NC, num_subcores=NS),
    compiler_params=pltpu.CompilerParams(needs_layout_passes=False),
    scratch_types=[
        pltpu.VMEM((BC,), jnp.int32),
        pltpu.VMEM((BC, H), jnp.float32),
        pltpu.VMEM_SHARED((NP, H), jnp.float32),
    ],
)(_cnt_body)


def _bn_body(acc_ref, cnt_ref, den_ref, gamma_ref, beta_ref, out_ref, *, relu):
    summed = jnp.concatenate(
        [acc_ref[0, :N, :], acc_ref[1, :N, :]], axis=1)
    cnt = cnt_ref[0, :N, 0:1] + cnt_ref[1, :N, 0:1]
    t = summed / jnp.clip(cnt, 1.0, None) + den_ref[...]
    mean = jnp.mean(t, axis=0, keepdims=True)
    var = jnp.mean((t - mean) ** 2, axis=0, keepdims=True)
    h = (t - mean) * lax.rsqrt(var + 1e-5) * gamma_ref[...] + beta_ref[...]
    if relu:
        h = jnp.maximum(h, 0.0)
    out_ref[...] = h


def _bn_stage(acc, cnt, den, gamma, beta, relu):
    return pl.pallas_call(
        functools.partial(_bn_body, relu=relu),
        grid=(1,),
        in_specs=[
            pl.BlockSpec((NC, NP, H), lambda i: (0, 0, 0)),
            pl.BlockSpec((NC, NP, H), lambda i: (0, 0, 0)),
            pl.BlockSpec((N, C), lambda i: (0, 0)),
            pl.BlockSpec((1, C), lambda i: (0, 0)),
            pl.BlockSpec((1, C), lambda i: (0, 0)),
        ],
        out_specs=pl.BlockSpec((N, C), lambda i: (0, 0)),
        out_shape=jax.ShapeDtypeStruct((N, C), jnp.float32),
    )(acc, cnt, den, gamma.reshape(1, C), beta.reshape(1, C))


def _layer(x, rec, cnt, g, mu, sigma, root, bias, lin, gamma, beta, relu):
    xg, den = _dense_stage(x, g, root, lin, bias)
    acc = _sc_conv(xg, rec, mu.reshape(K * DIM), sigma.reshape(K * DIM))
    return _bn_stage(acc, cnt, den, gamma, beta, relu)


def kernel(vals, edges, pseudo, g0, mu0, sigma0, root0, bias0, lin0, gamma0,
           beta0, g1, mu1, sigma1, root1, bias1, lin1, gamma1, beta1):
    src = edges[0]
    dst = edges[1]
    pbits = lax.bitcast_convert_type(pseudo.T, jnp.int32)
    rec = jnp.concatenate([src[None], dst[None], pbits], axis=0)
    rec = rec.reshape(6, E // B, B).transpose(1, 0, 2).reshape(E // B * REC)
    cnt = _sc_cnt(dst)
    h = _layer(vals, rec, cnt, g0, mu0, sigma0, root0, bias0, lin0,
               gamma0, beta0, True)
    h = _layer(h, rec, cnt, g1, mu1, sigma1, root1, bias1, lin1,
               gamma1, beta1, False)
    return h


# speedup vs baseline: 7.1800x; 1.0370x over previous
import functools

import jax
import jax.numpy as jnp
from jax import lax
from jax.experimental import pallas as pl
from jax.experimental.pallas import tpu as pltpu
from jax.experimental.pallas import tpu_sc as plsc

N = 10000
E = 160000
K = 4
DIM = 4
C = 256
H = 128
NC = 2
NS = 16
NW = NC * NS
B = 80
EPT = E // NS
NCHUNK = EPT // B
NP = 10240
HR = NP // H
STRIPE = NP // NS
ZR = 128
EPW = E // NW

_EPS = 1e-15
_ROW_BLK = 400


def _dense_body(x_ref, gcat_ref, root_ref, lin_ref, bias_ref, xg_ref, den_ref):
    x = x_ref[...]
    y = lax.dot_general(x, gcat_ref[...], (((1,), (1,)), ((), ())),
                        preferred_element_type=jnp.float32)
    y = y.reshape(_ROW_BLK, NC, 2, 2, 2, 64)
    lo = lax.bitcast_convert_type(
        y[:, :, :, :, 0, :].astype(jnp.bfloat16), jnp.uint16)
    hi = lax.bitcast_convert_type(
        y[:, :, :, :, 1, :].astype(jnp.bfloat16), jnp.uint16)
    w = lo.astype(jnp.uint32) | (hi.astype(jnp.uint32) << 16)
    xg_ref[...] = lax.bitcast_convert_type(w, jnp.float32).reshape(
        _ROW_BLK, NC, 2, H)
    den = lax.dot_general(x, root_ref[...], (((1,), (1,)), ((), ())),
                          preferred_element_type=jnp.float32)
    den += lax.dot_general(x, lin_ref[...], (((1,), (1,)), ((), ())),
                           preferred_element_type=jnp.float32)
    den_ref[...] = den + bias_ref[...]


def _dense_stage(x, g, root, lin, bias):
    gcat = (g.reshape(C, 2, 2, NC, H).transpose(3, 0, 1, 2, 4)
            .reshape(NC, C, 2, 2 * H))
    xg, den = pl.pallas_call(
        _dense_body,
        grid=(N // _ROW_BLK,),
        in_specs=[
            pl.BlockSpec((_ROW_BLK, C), lambda i: (i, 0)),
            pl.BlockSpec((NC, C, 2, 2 * H), lambda i: (0, 0, 0, 0)),
            pl.BlockSpec((C, C), lambda i: (0, 0)),
            pl.BlockSpec((C, C), lambda i: (0, 0)),
            pl.BlockSpec((1, C), lambda i: (0, 0)),
        ],
        out_specs=[
            pl.BlockSpec((_ROW_BLK, NC, 2, H), lambda i: (i, 0, 0, 0)),
            pl.BlockSpec((_ROW_BLK, C), lambda i: (i, 0)),
        ],
        out_shape=[
            jax.ShapeDtypeStruct((N, NC, 2, H), jnp.float32),
            jax.ShapeDtypeStruct((N, C), jnp.float32),
        ],
    )(x, gcat, root, lin, bias.reshape(1, C))
    return xg.reshape(N * NC * 2, H), den


REC = 6 * B


def _sc_body(xg_hbm, rec_hbm, mu_hbm, sg_hbm, out_hbm,
             mu_v, w_v, idxa_v, idxb_v, dsta_v, dstb_v, gau_v, rec_v,
             r0a_v, r1a_v, r0b_v, r1b_v, shared, semr, sems):
    cid = lax.axis_index("c")
    sid = lax.axis_index("s")

    pltpu.sync_copy(mu_hbm, mu_v)
    pltpu.sync_copy(sg_hbm, w_v)
    s = w_v[...]
    muv = mu_v[...]
    wv = -0.5 / (_EPS + s * s)

    def zrow(r, _):
        for i in range(H // 16):
            r0a_v[r, pl.ds(i * 16, 16)] = jnp.zeros((16,), jnp.float32)
        return _
    lax.fori_loop(0, B, zrow, 0)
    for i in range(STRIPE // B):
        pltpu.sync_copy(r0a_v, shared.at[pl.ds(sid * STRIPE + i * B, B)])

    plsc.subcore_barrier()

    bufs = ((r0a_v, r1a_v), (r0b_v, r1b_v))
    idxs = (idxa_v, idxb_v)
    dsts = (dsta_v, dstb_v)
    ifmt = plsc.PackFormat.INTERLEAVED

    def decode_fire(cc, pb):
        rb = pb * REC
        for i in range(B // 16):
            sl = pl.ds(i * 16, 16)
            i0 = rec_v[pl.ds(rb + i * 16, 16)] * 4 + cid * 2
            idxs[pb][pl.ds(i * 16, 16)] = i0
            idxs[pb][pl.ds(B + i * 16, 16)] = i0 + 1
            dsts[pb][sl] = rec_v[pl.ds(rb + B + i * 16, 16)]
        for h in range(2):
            pltpu.async_copy(xg_hbm.at[idxs[pb].at[pl.ds(h * B, B)]],
                             bufs[pb][h], sems.at[2 * pb + h])

    pltpu.sync_copy(rec_hbm.at[pl.ds(sid * NCHUNK * REC, REC)],
                    rec_v.at[pl.ds(0, REC)])
    decode_fire(0, 0)
    pltpu.async_copy(rec_hbm.at[pl.ds((sid * NCHUNK + 1) * REC, REC)],
                     rec_v.at[pl.ds(REC, REC)], semr)

    def stage(c, pb):
        qb = 1 - pb
        cur0, cur1 = bufs[pb]

        @pl.when(c < NCHUNK - 1)
        def _df():
            pltpu.make_async_copy(rec_hbm.at[pl.ds(0, REC)],
                                  rec_v.at[pl.ds(qb * REC, REC)], semr).wait()
            decode_fire(c + 1, qb)

        rb = pb * REC
        for k in range(K):
            for i in range(B // 16):
                sl = pl.ds(i * 16, 16)
                acc = jnp.zeros((16,), jnp.float32)
                for d in range(DIM):
                    t = plsc.bitcast(
                        rec_v[pl.ds(rb + (2 + d) * B + i * 16, 16)],
                        jnp.float32) - muv[k * DIM + d]
                    acc += t * t * wv[k * DIM + d]
                gau_v[pl.ds(k * B + i * 16, 16)] = jnp.exp(acc)

        @pl.when(c < NCHUNK - 2)
        def _pf():
            nxt = sid * NCHUNK + c + 2
            pltpu.async_copy(rec_hbm.at[pl.ds(nxt * REC, REC)],
                             rec_v.at[pl.ds(pb * REC, REC)], semr)

        for h in range(2):
            pltpu.make_async_copy(xg_hbm.at[idxs[pb].at[pl.ds(h * B, B)]],
                                  bufs[pb][h], sems.at[2 * pb + h]).wait()

        def edge(b, _):
            bs = jnp.full((16,), b, jnp.int32)
            gp = []
            for k in range(K):
                gk = plsc.load_gather(gau_v, [bs + k * B])
                gp.append(plsc.pack(gk, gk, format=ifmt))
            for q in range(H // 32):
                v = (gp[0] * plsc.bitcast(cur0[b, pl.ds(q * 16, 16)],
                                          jnp.bfloat16)
                     + gp[1] * plsc.bitcast(cur0[b, pl.ds(64 + q * 16, 16)],
                                            jnp.bfloat16)
                     + gp[2] * plsc.bitcast(cur1[b, pl.ds(q * 16, 16)],
                                            jnp.bfloat16)
                     + gp[3] * plsc.bitcast(cur1[b, pl.ds(64 + q * 16, 16)],
                                            jnp.bfloat16))
                va, vb = plsc.unpack(v, format=ifmt)
                cur0[b, pl.ds(q * 16, 16)] = va
                cur0[b, pl.ds(64 + q * 16, 16)] = vb
            return _
        lax.fori_loop(0, B, edge, 0)

        pltpu.sync_copy(cur0, shared.at[dsts[pb]], add=True)

    def chunk(c, _):
        p = lax.rem(c, 2)

        @pl.when(p == 0)
        def _even():
            stage(c, 0)

        @pl.when(p == 1)
        def _odd():
            stage(c, 1)
        return _
    lax.fori_loop(0, NCHUNK, chunk, 0)

    plsc.subcore_barrier()
    pltpu.sync_copy(shared.at[pl.ds(sid * STRIPE, STRIPE)],
                    out_hbm.at[cid, pl.ds(sid * STRIPE, STRIPE)])


_sc_conv = functools.partial(
    pl.kernel,
    out_type=jax.ShapeDtypeStruct((NC, NP, H), jnp.float32),
    mesh=plsc.VectorSubcoreMesh(core_axis_name="c", subcore_axis_name="s", num_cores=NC, num_subcores=NS),
    compiler_params=pltpu.CompilerParams(needs_layout_passes=False),
    scratch_types=[
        pltpu.VMEM((16,), jnp.float32),
        pltpu.VMEM((16,), jnp.float32),
        pltpu.VMEM((2 * B,), jnp.int32),
        pltpu.VMEM((2 * B,), jnp.int32),
        pltpu.VMEM((B,), jnp.int32),
        pltpu.VMEM((B,), jnp.int32),
        pltpu.VMEM((K * B,), jnp.float32),
        pltpu.VMEM((2 * REC,), jnp.int32),
        pltpu.VMEM((B, H), jnp.float32),
        pltpu.VMEM((B, H), jnp.float32),
        pltpu.VMEM((B, H), jnp.float32),
        pltpu.VMEM((B, H), jnp.float32),
        pltpu.VMEM_SHARED((NP, H), jnp.float32),
        pltpu.SemaphoreType.DMA,
        pltpu.SemaphoreType.DMA((4,)),
    ],
)(_sc_body)


BC = 200
EPC = E // NC
EPTC = EPC // NS


def _cnt_body(dst_hbm, out_hbm, dst_v, one_v, shared):
    cid = lax.axis_index("c")
    sid = lax.axis_index("s")

    def zrow(r, _):
        for i in range(H // 16):
            one_v[r, pl.ds(i * 16, 16)] = jnp.zeros((16,), jnp.float32)
        return _
    lax.fori_loop(0, BC, zrow, 0)
    for i in range(STRIPE // BC):
        pltpu.sync_copy(one_v, shared.at[pl.ds(sid * STRIPE + i * BC, BC)])
    rem = STRIPE - (STRIPE // BC) * BC
    if rem:
        pltpu.sync_copy(
            one_v.at[pl.ds(0, rem)],
            shared.at[pl.ds(sid * STRIPE + (STRIPE // BC) * BC, rem)])

    def orow(r, _):
        for i in range(H // 16):
            one_v[r, pl.ds(i * 16, 16)] = jnp.ones((16,), jnp.float32)
        return _
    lax.fori_loop(0, BC, orow, 0)

    plsc.subcore_barrier()

    def chunk(c, _):
        ebase = cid * EPC + sid * EPTC + c * BC
        pltpu.sync_copy(dst_hbm.at[pl.ds(ebase, BC)], dst_v)
        pltpu.sync_copy(one_v, shared.at[dst_v], add=True)
        return _
    lax.fori_loop(0, EPTC // BC, chunk, 0)

    plsc.subcore_barrier()
    pltpu.sync_copy(shared.at[pl.ds(sid * STRIPE, STRIPE)],
                    out_hbm.at[cid, pl.ds(sid * STRIPE, STRIPE)])


_sc_cnt = functools.partial(
    pl.kernel,
    out_type=jax.ShapeDtypeStruct((NC, NP, H), jnp.float32),
    mesh=plsc.VectorSubcoreMesh(core_axis_name="c", subcore_axis_name="s", num_cores=NC, num_subcores=NS),
    compiler_params=pltpu.CompilerParams(needs_layout_passes=False),
    scratch_types=[
        pltpu.VMEM((BC,), jnp.int32),
        pltpu.VMEM((BC, H), jnp.float32),
        pltpu.VMEM_SHARED((NP, H), jnp.float32),
    ],
)(_cnt_body)


def _bn_body(acc_ref, cnt_ref, den_ref, gamma_ref, beta_ref, out_ref, *, relu):
    summed = jnp.concatenate(
        [acc_ref[0, :N, :], acc_ref[1, :N, :]], axis=1)
    cnt = cnt_ref[0, :N, 0:1] + cnt_ref[1, :N, 0:1]
    t = summed / jnp.clip(cnt, 1.0, None) + den_ref[...]
    mean = jnp.mean(t, axis=0, keepdims=True)
    var = jnp.mean((t - mean) ** 2, axis=0, keepdims=True)
    h = (t - mean) * lax.rsqrt(var + 1e-5) * gamma_ref[...] + beta_ref[...]
    if relu:
        h = jnp.maximum(h, 0.0)
    out_ref[...] = h


def _bn_stage(acc, cnt, den, gamma, beta, relu):
    return pl.pallas_call(
        functools.partial(_bn_body, relu=relu),
        grid=(1,),
        in_specs=[
            pl.BlockSpec((NC, NP, H), lambda i: (0, 0, 0)),
            pl.BlockSpec((NC, NP, H), lambda i: (0, 0, 0)),
            pl.BlockSpec((N, C), lambda i: (0, 0)),
            pl.BlockSpec((1, C), lambda i: (0, 0)),
            pl.BlockSpec((1, C), lambda i: (0, 0)),
        ],
        out_specs=pl.BlockSpec((N, C), lambda i: (0, 0)),
        out_shape=jax.ShapeDtypeStruct((N, C), jnp.float32),
    )(acc, cnt, den, gamma.reshape(1, C), beta.reshape(1, C))


def _layer(x, rec, cnt, g, mu, sigma, root, bias, lin, gamma, beta, relu):
    xg, den = _dense_stage(x, g, root, lin, bias)
    acc = _sc_conv(xg, rec, mu.reshape(K * DIM), sigma.reshape(K * DIM))
    return _bn_stage(acc, cnt, den, gamma, beta, relu)


def kernel(vals, edges, pseudo, g0, mu0, sigma0, root0, bias0, lin0, gamma0,
           beta0, g1, mu1, sigma1, root1, bias1, lin1, gamma1, beta1):
    src = edges[0]
    dst = edges[1]
    pbits = lax.bitcast_convert_type(pseudo.T, jnp.int32)
    rec = jnp.concatenate([src[None], dst[None], pbits], axis=0)
    rec = rec.reshape(6, E // B, B).transpose(1, 0, 2).reshape(E // B * REC)
    cnt = _sc_cnt(dst)
    h = _layer(vals, rec, cnt, g0, mu0, sigma0, root0, bias0, lin0,
               gamma0, beta0, True)
    h = _layer(h, rec, cnt, g1, mu1, sigma1, root1, bias1, lin1,
               gamma1, beta1, False)
    return h
